# baseline (speedup 1.0000x reference)
import functools

import jax
import jax.numpy as jnp
from jax import lax
from jax.experimental import pallas as pl
from jax.experimental.pallas import tpu as pltpu
from jax.experimental.pallas import tpu_sc as plsc

N, D, K, G = 10000, 512, 4, 64
NPAD = 10240
BLK = 1024
NBLK = NPAD // BLK
F = 2 * D + 128

NC, NS = 2, 16
NW = NC * NS
R = NPAD // NW
CH = 32
NCH = R // CH
GACC = 65
NF16 = F // 16


def _encoder_kernel(x_ref, w1_ref, b1_ref, w2_ref, b2_ref, wt_ref, bt_ref,
                    feat_ref):
    x = x_ref[...]
    h = jnp.maximum(
        lax.dot_general(x, w1_ref[...], (((1,), (1,)), ((), ())),
                        preferred_element_type=jnp.float32) + b1_ref[...], 0.0)
    xe = lax.dot_general(h, w2_ref[...], (((1,), (1,)), ((), ())),
                         preferred_element_type=jnp.float32) + b2_ref[...]
    w = jnp.sum(xe * wt_ref[...], axis=1, keepdims=True) + bt_ref[0]
    feat_ref[...] = jnp.concatenate(
        [xe * xe, xe * w, jnp.broadcast_to(w * w, (BLK, 128))], axis=1)


def _node_features(xp, W1, b1, W2, b2, Wt, bt, interpret=False):
    return pl.pallas_call(
        _encoder_kernel,
        grid=(NBLK,),
        in_specs=[
            pl.BlockSpec((BLK, D), lambda i: (i, 0)),
            pl.BlockSpec((D, D), lambda i: (0, 0)),
            pl.BlockSpec((1, D), lambda i: (0, 0)),
            pl.BlockSpec((D, D), lambda i: (0, 0)),
            pl.BlockSpec((1, D), lambda i: (0, 0)),
            pl.BlockSpec((1, D), lambda i: (0, 0)),
            pl.BlockSpec(memory_space=pltpu.SMEM),
        ],
        out_specs=pl.BlockSpec((BLK, F), lambda i: (i, 0)),
        out_shape=jax.ShapeDtypeStruct((NPAD, F), jnp.float32),
        interpret=interpret,
    )(xp, W1, b1.reshape(1, D), W2, b2.reshape(1, D), Wt, bt.reshape(1,))


def _sc_segment_sums(feat, ids2, zz):
    mesh = plsc.VectorSubcoreMesh(core_axis_name="c", subcore_axis_name="s",
                                  num_cores=NC, num_subcores=NS)

    @functools.partial(
        pl.kernel,
        out_type=jax.ShapeDtypeStruct((NW, GACC * F), jnp.float32),
        mesh=mesh,
        compiler_params=pltpu.CompilerParams(needs_layout_passes=False),
        scratch_types=[
            pltpu.VMEM((CH * 16,), jnp.int32),
            pltpu.VMEM((CH * F,), jnp.float32),
            pltpu.VMEM((GACC * F,), jnp.float32),
        ],
    )
    def run(feat_hbm, ids_hbm, zz_hbm, out_hbm, idc_v, buf_v, acc_v):
        cid = lax.axis_index("c")
        sid = lax.axis_index("s")
        wid = cid * NS + sid

        pltpu.sync_copy(zz_hbm, acc_v)
        row0 = wid * R
        col_iota = lax.iota(jnp.int32, 16)

        def chunk_body(c, carry):
            pltpu.sync_copy(ids_hbm.at[pl.ds((row0 + c * CH) * 16, CH * 16)],
                            idc_v)
            pltpu.sync_copy(feat_hbm.at[pl.ds((row0 + c * CH) * F, CH * F)],
                            buf_v)

            def row_body(r, carry2):
                gvec = idc_v[pl.ds(r * 16, 16)]
                base = gvec * F + col_iota
                for j in range(NF16):
                    x = buf_v[pl.ds(r * F + 16 * j, 16)]
                    plsc.addupdate_scatter(acc_v, [base + 16 * j], x)
                return carry2

            return lax.fori_loop(0, CH, row_body, carry)

        lax.fori_loop(0, NCH, chunk_body, 0)
        pltpu.sync_copy(acc_v, out_hbm.at[wid])

    return run(feat, ids2, zz)


def _head_kernel(pr_ref, cent_ref, beta_ref, wm1_ref, bm1_ref,
                 wm2_ref, bm2_ref, out_ref):
    sums = pr_ref[0:G, :]
    for j in range(1, NW):
        sums = sums + pr_ref[j * GACC:j * GACC + G, :]
    s2 = sums[:, :D]
    sxw = sums[:, D:2 * D]
    sw2 = sums[:, 2 * D:2 * D + 1]
    inv_beta = 1.0 / beta_ref[0, 0]
    parts = []
    for k in range(K):
        ck = cent_ref[0, k]
        seg = s2 - (2.0 * ck) * sxw + (ck * ck) * sw2
        parts.append(jnp.exp(-jnp.sqrt(jnp.maximum(seg, 0.0)) * inv_beta))
    r4 = jnp.concatenate(parts, axis=1)
    ssum = jnp.sum(r4 * r4, axis=1, keepdims=True)
    r4 = r4 * (1.0 / jnp.maximum(jnp.sqrt(ssum), 1e-12))
    hh = jnp.maximum(
        lax.dot_general(r4, wm1_ref[...], (((1,), (1,)), ((), ())),
                        preferred_element_type=jnp.float32) + bm1_ref[...], 0.0)
    out_ref[...] = lax.dot_general(hh, wm2_ref[...], (((1,), (1,)), ((), ())),
                                   preferred_element_type=jnp.float32) + bm2_ref[...]


def _head(partials2, centers, beta, Wm1, bm1, Wm2, bm2, interpret=False):
    KD = K * D
    return pl.pallas_call(
        _head_kernel,
        in_specs=[
            pl.BlockSpec(memory_space=pltpu.VMEM),
            pl.BlockSpec(memory_space=pltpu.SMEM),
            pl.BlockSpec(memory_space=pltpu.SMEM),
            pl.BlockSpec(memory_space=pltpu.VMEM),
            pl.BlockSpec(memory_space=pltpu.VMEM),
            pl.BlockSpec(memory_space=pltpu.VMEM),
            pl.BlockSpec(memory_space=pltpu.VMEM),
        ],
        out_shape=jax.ShapeDtypeStruct((G, D), jnp.float32),
        interpret=interpret,
    )(partials2, centers, beta, Wm1, bm1.reshape(1, KD), Wm2,
      bm2.reshape(1, D))


def kernel(x, batch, W1, b1, W2, b2, Wt, bt, centers, beta, Wm1, bm1, Wm2,
           bm2, interpret=False):
    xp = jnp.pad(x, ((0, NPAD - N), (0, 0)))
    idp = jnp.concatenate(
        [batch.astype(jnp.int32), jnp.full((NPAD - N,), G, jnp.int32)])
    ids2 = jnp.broadcast_to(idp[:, None], (NPAD, 16)).reshape(NPAD * 16)
    feat = _node_features(xp, W1, b1, W2, b2, Wt, bt, interpret=interpret)
    zz = jnp.zeros((GACC * F,), jnp.float32)
    partials = _sc_segment_sums(feat.reshape(NPAD * F), ids2, zz)
    partials2 = partials.reshape(NW * GACC, F)
    return _head(partials2, centers, beta, Wm1, bm1, Wm2, bm2,
                 interpret=interpret)


# speedup vs baseline: 1.0786x; 1.0786x over previous
import functools

import jax
import jax.numpy as jnp
from jax import lax
from jax.experimental import pallas as pl
from jax.experimental.pallas import tpu as pltpu
from jax.experimental.pallas import tpu_sc as plsc

N, D, K, G = 10000, 512, 4, 64
NPAD = 10240
BLK = 1024
NBLK = NPAD // BLK
F = 2 * D + 128

NC, NS = 2, 16
NW = NC * NS
R = NPAD // NW
CH = 16
NCH = R // CH
GACC = 65
NF16 = F // 16


def _encoder_kernel(x_ref, w1_ref, b1_ref, w2_ref, b2_ref, wt_ref, bt_ref,
                    feat_ref):
    x = x_ref[...]
    h = jnp.maximum(
        lax.dot_general(x, w1_ref[...], (((1,), (1,)), ((), ())),
                        preferred_element_type=jnp.float32) + b1_ref[...], 0.0)
    xe = lax.dot_general(h, w2_ref[...], (((1,), (1,)), ((), ())),
                         preferred_element_type=jnp.float32) + b2_ref[...]
    w = jnp.sum(xe * wt_ref[...], axis=1, keepdims=True) + bt_ref[0]
    feat_ref[...] = jnp.concatenate(
        [xe * xe, xe * w, jnp.broadcast_to(w * w, (BLK, 128))], axis=1)


def _node_features(xp, W1, b1, W2, b2, Wt, bt, interpret=False):
    return pl.pallas_call(
        _encoder_kernel,
        grid=(NBLK,),
        in_specs=[
            pl.BlockSpec((BLK, D), lambda i: (i, 0)),
            pl.BlockSpec((D, D), lambda i: (0, 0)),
            pl.BlockSpec((1, D), lambda i: (0, 0)),
            pl.BlockSpec((D, D), lambda i: (0, 0)),
            pl.BlockSpec((1, D), lambda i: (0, 0)),
            pl.BlockSpec((1, D), lambda i: (0, 0)),
            pl.BlockSpec(memory_space=pltpu.SMEM),
        ],
        out_specs=pl.BlockSpec((BLK, F), lambda i: (i, 0)),
        out_shape=jax.ShapeDtypeStruct((NPAD, F), jnp.float32),
        interpret=interpret,
    )(xp, W1, b1.reshape(1, D), W2, b2.reshape(1, D), Wt, bt.reshape(1,))


def _sc_segment_sums(feat, ids2, zz):
    mesh = plsc.VectorSubcoreMesh(core_axis_name="c", subcore_axis_name="s",
                                  num_cores=NC, num_subcores=NS)

    @functools.partial(
        pl.kernel,
        out_type=jax.ShapeDtypeStruct((NW, GACC * F), jnp.float32),
        mesh=mesh,
        compiler_params=pltpu.CompilerParams(needs_layout_passes=False),
        scratch_types=[
            pltpu.VMEM((CH * 16,), jnp.int32),
            pltpu.VMEM((CH * 16,), jnp.int32),
            pltpu.VMEM((CH * F,), jnp.float32),
            pltpu.VMEM((CH * F,), jnp.float32),
            pltpu.VMEM((GACC * F,), jnp.float32),
            pltpu.SemaphoreType.DMA,
        ],
    )
    def run(feat_hbm, ids_hbm, zz_hbm, out_hbm, idc0, idc1, buf0, buf1,
            acc_v, sem):
        cid = lax.axis_index("c")
        sid = lax.axis_index("s")
        wid = cid * NS + sid

        pltpu.sync_copy(zz_hbm, acc_v)
        row0 = wid * R
        col_iota = lax.iota(jnp.int32, 16)
        idc = (idc0, idc1)
        buf = (buf0, buf1)

        def issue(c, b):
            return (
                pltpu.async_copy(
                    ids_hbm.at[pl.ds((row0 + c * CH) * 16, CH * 16)],
                    idc[b], sem),
                pltpu.async_copy(
                    feat_hbm.at[pl.ds((row0 + c * CH) * F, CH * F)],
                    buf[b], sem),
            )

        def process(b):
            def row_body(r, carry2):
                gvec = idc[b][pl.ds(r * 16, 16)]
                base = gvec * F + col_iota
                for j in range(NF16):
                    x = buf[b][pl.ds(r * F + 16 * j, 16)]
                    plsc.addupdate_scatter(acc_v, [base + 16 * j], x)
                return carry2

            lax.fori_loop(0, CH, row_body, 0)

        pending = issue(0, 0)
        for c in range(NCH):
            for h in pending:
                h.wait()
            if c + 1 < NCH:
                pending = issue(c + 1, (c + 1) % 2)
            process(c % 2)

        pltpu.sync_copy(acc_v, out_hbm.at[wid])

    return run(feat, ids2, zz)


def _head_kernel(pr_ref, cent_ref, beta_ref, wm1_ref, bm1_ref,
                 wm2_ref, bm2_ref, out_ref):
    sums = pr_ref[0:G, :]
    for j in range(1, NW):
        sums = sums + pr_ref[j * GACC:j * GACC + G, :]
    s2 = sums[:, :D]
    sxw = sums[:, D:2 * D]
    sw2 = sums[:, 2 * D:2 * D + 1]
    inv_beta = 1.0 / beta_ref[0, 0]
    parts = []
    for k in range(K):
        ck = cent_ref[0, k]
        seg = s2 - (2.0 * ck) * sxw + (ck * ck) * sw2
        parts.append(jnp.exp(-jnp.sqrt(jnp.maximum(seg, 0.0)) * inv_beta))
    r4 = jnp.concatenate(parts, axis=1)
    ssum = jnp.sum(r4 * r4, axis=1, keepdims=True)
    r4 = r4 * (1.0 / jnp.maximum(jnp.sqrt(ssum), 1e-12))
    hh = jnp.maximum(
        lax.dot_general(r4, wm1_ref[...], (((1,), (1,)), ((), ())),
                        preferred_element_type=jnp.float32) + bm1_ref[...], 0.0)
    out_ref[...] = lax.dot_general(hh, wm2_ref[...], (((1,), (1,)), ((), ())),
                                   preferred_element_type=jnp.float32) + bm2_ref[...]


def _head(partials2, centers, beta, Wm1, bm1, Wm2, bm2, interpret=False):
    KD = K * D
    return pl.pallas_call(
        _head_kernel,
        in_specs=[
            pl.BlockSpec(memory_space=pltpu.VMEM),
            pl.BlockSpec(memory_space=pltpu.SMEM),
            pl.BlockSpec(memory_space=pltpu.SMEM),
            pl.BlockSpec(memory_space=pltpu.VMEM),
            pl.BlockSpec(memory_space=pltpu.VMEM),
            pl.BlockSpec(memory_space=pltpu.VMEM),
            pl.BlockSpec(memory_space=pltpu.VMEM),
        ],
        out_shape=jax.ShapeDtypeStruct((G, D), jnp.float32),
        interpret=interpret,
    )(partials2, centers, beta, Wm1, bm1.reshape(1, KD), Wm2,
      bm2.reshape(1, D))


def kernel(x, batch, W1, b1, W2, b2, Wt, bt, centers, beta, Wm1, bm1, Wm2,
           bm2, interpret=False):
    xp = jnp.pad(x, ((0, NPAD - N), (0, 0)))
    idp = jnp.concatenate(
        [batch.astype(jnp.int32), jnp.full((NPAD - N,), G, jnp.int32)])
    ids2 = jnp.broadcast_to(idp[:, None], (NPAD, 16)).reshape(NPAD * 16)
    feat = _node_features(xp, W1, b1, W2, b2, Wt, bt, interpret=interpret)
    zz = jnp.zeros((GACC * F,), jnp.float32)
    partials = _sc_segment_sums(feat.reshape(NPAD * F), ids2, zz)
    partials2 = partials.reshape(NW * GACC, F)
    return _head(partials2, centers, beta, Wm1, bm1, Wm2, bm2,
                 interpret=interpret)


# speedup vs baseline: 1.1004x; 1.0202x over previous
import functools

import jax
import jax.numpy as jnp
from jax import lax
from jax.experimental import pallas as pl
from jax.experimental.pallas import tpu as pltpu
from jax.experimental.pallas import tpu_sc as plsc

N, D, K, G = 10000, 512, 4, 64
NPAD = 10240
BLK = 1024
NBLK = NPAD // BLK
F = 2 * D + 128

NC, NS = 2, 16
NW = NC * NS
R = NPAD // NW
CH = 16
NCH = R // CH
GACC = 65
NF16 = F // 16


def _encoder_kernel(x_ref, w1_ref, b1_ref, w2_ref, b2_ref, wt_ref, bt_ref,
                    feat_ref):
    x = x_ref[...]
    h = jnp.maximum(
        lax.dot_general(x, w1_ref[...], (((1,), (1,)), ((), ())),
                        preferred_element_type=jnp.float32) + b1_ref[...], 0.0)
    xe = lax.dot_general(h, w2_ref[...], (((1,), (1,)), ((), ())),
                         preferred_element_type=jnp.float32) + b2_ref[...]
    w = jnp.sum(xe * wt_ref[...], axis=1, keepdims=True) + bt_ref[0]
    feat_ref[...] = jnp.concatenate(
        [xe * xe, xe * w, jnp.broadcast_to(w * w, (BLK, 128))], axis=1)


def _node_features(xp, W1, b1, W2, b2, Wt, bt, interpret=False):
    return pl.pallas_call(
        _encoder_kernel,
        grid=(NBLK,),
        in_specs=[
            pl.BlockSpec((BLK, D), lambda i: (i, 0)),
            pl.BlockSpec((D, D), lambda i: (0, 0)),
            pl.BlockSpec((1, D), lambda i: (0, 0)),
            pl.BlockSpec((D, D), lambda i: (0, 0)),
            pl.BlockSpec((1, D), lambda i: (0, 0)),
            pl.BlockSpec((1, D), lambda i: (0, 0)),
            pl.BlockSpec(memory_space=pltpu.SMEM),
        ],
        out_specs=pl.BlockSpec((BLK, F), lambda i: (i, 0)),
        out_shape=jax.ShapeDtypeStruct((NPAD, F), jnp.float32),
        interpret=interpret,
    )(xp, W1, b1.reshape(1, D), W2, b2.reshape(1, D), Wt, bt.reshape(1,))


def _sc_segment_sums(feat, ids2, zz):
    mesh = plsc.VectorSubcoreMesh(core_axis_name="c", subcore_axis_name="s",
                                  num_cores=NC, num_subcores=NS)

    @functools.partial(
        pl.kernel,
        out_type=jax.ShapeDtypeStruct((NW, GACC * F), jnp.float32),
        mesh=mesh,
        compiler_params=pltpu.CompilerParams(needs_layout_passes=False),
        scratch_types=[
            pltpu.VMEM((CH * 16,), jnp.int32),
            pltpu.VMEM((CH * 16,), jnp.int32),
            pltpu.VMEM((CH * F,), jnp.float32),
            pltpu.VMEM((CH * F,), jnp.float32),
            pltpu.VMEM((GACC * F,), jnp.float32),
            pltpu.SemaphoreType.DMA,
        ],
    )
    def run(feat_hbm, ids_hbm, zz_hbm, out_hbm, idc0, idc1, buf0, buf1,
            acc_v, sem):
        cid = lax.axis_index("c")
        sid = lax.axis_index("s")
        wid = cid * NS + sid

        pltpu.sync_copy(zz_hbm, acc_v)
        row0 = wid * R
        col_iota = lax.iota(jnp.int32, 16)
        idc = (idc0, idc1)
        buf = (buf0, buf1)

        def issue(c, b):
            pltpu.async_copy(
                ids_hbm.at[pl.ds((row0 + c * CH) * 16, CH * 16)],
                idc[b], sem)
            pltpu.async_copy(
                feat_hbm.at[pl.ds((row0 + c * CH) * F, CH * F)],
                buf[b], sem)

        def wait(b):
            pltpu.make_async_copy(
                ids_hbm.at[pl.ds(0, CH * 16)], idc[b], sem).wait()
            pltpu.make_async_copy(
                feat_hbm.at[pl.ds(0, CH * F)], buf[b], sem).wait()

        def process(b):
            def row_body(r, carry2):
                gvec = idc[b][pl.ds(r * 16, 16)]
                base = gvec * F + col_iota
                for j in range(NF16):
                    x = buf[b][pl.ds(r * F + 16 * j, 16)]
                    plsc.addupdate_scatter(
                        acc_v.at[pl.ds(16 * j, GACC * F - 16 * j)],
                        [base], x)
                return carry2

            lax.fori_loop(0, CH, row_body, 0)

        issue(0, 0)
        issue(1, 1)

        def pair_body(t, carry):
            for b in range(2):
                c = 2 * t + b
                wait(b)
                process(b)
                issue(jnp.minimum(c + 2, NCH - 1), b)
            return carry

        lax.fori_loop(0, NCH // 2, pair_body, 0)
        wait(0)
        wait(1)

        pltpu.sync_copy(acc_v, out_hbm.at[wid])

    return run(feat, ids2, zz)


def _head_kernel(pr_ref, cent_ref, beta_ref, wm1_ref, bm1_ref,
                 wm2_ref, bm2_ref, out_ref):
    sums = pr_ref[0:G, :]
    for j in range(1, NW):
        sums = sums + pr_ref[j * GACC:j * GACC + G, :]
    s2 = sums[:, :D]
    sxw = sums[:, D:2 * D]
    sw2 = sums[:, 2 * D:2 * D + 1]
    inv_beta = 1.0 / beta_ref[0, 0]
    parts = []
    for k in range(K):
        ck = cent_ref[0, k]
        seg = s2 - (2.0 * ck) * sxw + (ck * ck) * sw2
        parts.append(jnp.exp(-jnp.sqrt(jnp.maximum(seg, 0.0)) * inv_beta))
    r4 = jnp.concatenate(parts, axis=1)
    ssum = jnp.sum(r4 * r4, axis=1, keepdims=True)
    r4 = r4 * (1.0 / jnp.maximum(jnp.sqrt(ssum), 1e-12))
    hh = jnp.maximum(
        lax.dot_general(r4, wm1_ref[...], (((1,), (1,)), ((), ())),
                        preferred_element_type=jnp.float32) + bm1_ref[...], 0.0)
    out_ref[...] = lax.dot_general(hh, wm2_ref[...], (((1,), (1,)), ((), ())),
                                   preferred_element_type=jnp.float32) + bm2_ref[...]


def _head(partials2, centers, beta, Wm1, bm1, Wm2, bm2, interpret=False):
    KD = K * D
    return pl.pallas_call(
        _head_kernel,
        in_specs=[
            pl.BlockSpec(memory_space=pltpu.VMEM),
            pl.BlockSpec(memory_space=pltpu.SMEM),
            pl.BlockSpec(memory_space=pltpu.SMEM),
            pl.BlockSpec(memory_space=pltpu.VMEM),
            pl.BlockSpec(memory_space=pltpu.VMEM),
            pl.BlockSpec(memory_space=pltpu.VMEM),
            pl.BlockSpec(memory_space=pltpu.VMEM),
        ],
        out_shape=jax.ShapeDtypeStruct((G, D), jnp.float32),
        interpret=interpret,
    )(partials2, centers, beta, Wm1, bm1.reshape(1, KD), Wm2,
      bm2.reshape(1, D))


def kernel(x, batch, W1, b1, W2, b2, Wt, bt, centers, beta, Wm1, bm1, Wm2,
           bm2, interpret=False):
    xp = jnp.pad(x, ((0, NPAD - N), (0, 0)))
    idp = jnp.concatenate(
        [batch.astype(jnp.int32), jnp.full((NPAD - N,), G, jnp.int32)])
    ids2 = jnp.broadcast_to(idp[:, None], (NPAD, 16)).reshape(NPAD * 16)
    feat = _node_features(xp, W1, b1, W2, b2, Wt, bt, interpret=interpret)
    zz = jnp.zeros((GACC * F,), jnp.float32)
    partials = _sc_segment_sums(feat.reshape(NPAD * F), ids2, zz)
    partials2 = partials.reshape(NW * GACC, F)
    return _head(partials2, centers, beta, Wm1, bm1, Wm2, bm2,
                 interpret=interpret)
